# baseline (speedup 1.0000x reference)
import functools

import jax
import jax.numpy as jnp
from jax import lax
from jax.experimental import pallas as pl
from jax.experimental.pallas import tpu as pltpu
from jax.experimental.pallas import tpu_sc as plsc

_P = 320000
_BS = 10000
_NV = 10000
_DIM = 128
_H = 4
_C = 32
_HC = _H * _C
_HH = 64

_NS = 16
_PW = _P // _NS
_E = 80
_NCHUNK = _PW // _E

_ACC_R = 5008
_RPT = 312
_DROW = 2 * _BS + 96

_ROWS_A = 1000
_ROWS_C = 1000


def _peproj_body(pe_ref, w_ref, b_ref, out_ref):
    full = (
        jnp.dot(pe_ref[...], w_ref[...], preferred_element_type=jnp.float32)
        + b_ref[...]
    )
    out_ref[0] = full[:, :_HH]
    out_ref[1] = full[:, _HH:]


def _pe_proj(pe_enc, W_r2, b_r):
    return pl.pallas_call(
        _peproj_body,
        grid=(_P // _ROWS_A,),
        in_specs=[
            pl.BlockSpec((_ROWS_A, 2 * _DIM), lambda i: (i, 0)),
            pl.BlockSpec((2 * _DIM, _HC), lambda i: (0, 0)),
            pl.BlockSpec((1, _HC), lambda i: (0, 0)),
        ],
        out_specs=pl.BlockSpec((2, _ROWS_A, _HH), lambda i: (0, i, 0)),
        out_shape=jax.ShapeDtypeStruct((2, _P, _HH), jnp.float32),
    )(pe_enc, W_r2, b_r.reshape(1, _HC))


def _pre_body(ex_ref, nx_ref, wl_ref, wr1_ref, bl_ref, g_ref, hn_ref):
    esum = ex_ref[:, :_DIM] + ex_ref[:, _DIM:]
    gfull = (
        jnp.dot(esum, wl_ref[...], preferred_element_type=jnp.float32)
        + 2.0 * bl_ref[...]
    )
    g_ref[0] = gfull[:, :_HH]
    g_ref[1] = gfull[:, _HH:]
    hfull = jnp.dot(nx_ref[...], wr1_ref[...], preferred_element_type=jnp.float32)
    hn_ref[0] = hfull[:, :_HH]
    hn_ref[1] = hfull[:, _HH:]


def _precompute(edge_x, node_x, W_l, W_r1, b_l):
    return pl.pallas_call(
        _pre_body,
        out_shape=(
            jax.ShapeDtypeStruct((2, _BS, _HH), jnp.float32),
            jax.ShapeDtypeStruct((2, _NV, _HH), jnp.float32),
        ),
    )(edge_x, node_x, W_l, W_r1, b_l.reshape(1, _HC))


_sc_mesh = plsc.VectorSubcoreMesh(core_axis_name="c", subcore_axis_name="s")

_GDN = lax.GatherDimensionNumbers(
    offset_dims=(), collapsed_slice_dims=(0,), start_index_map=(0,))


def _allsum16(v):
    for s in (8, 4, 2, 1):
        idx = (lax.iota(jnp.int32, 16) ^ s).reshape(16, 1)
        v = v + lax.gather(v, idx, _GDN, slice_sizes=(1,),
                           mode=lax.GatherScatterMode.PROMISE_IN_BOUNDS)
    return v


@functools.partial(
    pl.kernel,
    out_type=(
        jax.ShapeDtypeStruct((2, _ACC_R, _HC), jnp.float32),
        jax.ShapeDtypeStruct((2, _NS, _DROW), jnp.float32),
    ),
    mesh=_sc_mesh,
    scratch_types=[
        pltpu.VMEM((_E + 16,), jnp.int32),
        pltpu.VMEM((1, _E), jnp.int32),
        pltpu.VMEM((_E + 16,), jnp.int32),
        pltpu.VMEM((1, _E), jnp.int32),
        pltpu.VMEM((_E, _HH), jnp.float32),
        pltpu.VMEM((_E, _HC), jnp.float32),
        pltpu.VMEM((_E, _HC), jnp.float32),
        pltpu.VMEM((_E, _HC), jnp.float32),
        pltpu.VMEM((_DROW,), jnp.float32),
        pltpu.VMEM((_HH,), jnp.float32),
        pltpu.VMEM_SHARED((_ACC_R, _HC), jnp.float32),
        pltpu.SemaphoreType.DMA,
        pltpu.SemaphoreType.DMA,
        pltpu.SemaphoreType.DMA,
    ],
)
def _sc_edges(src_hbm, dst_hbm, pe_hbm, hn_hbm, g_hbm, att_hbm, zero_hbm, zerod_hbm,
              acc_hbm, den_hbm,
              src_v, sidx_v, dst_v, didx_v, pe_v, hn_v, g_v, msg_v, den_v, att_v,
              acc_sh, sem0, sem1, sem2):
    cid = lax.axis_index("c")
    sid = lax.axis_index("s")

    pltpu.sync_copy(zero_hbm, acc_sh.at[pl.ds(sid * _RPT, _RPT)])

    @pl.when(sid == 0)
    def _init_tail():
        pltpu.sync_copy(zero_hbm.at[pl.ds(0, 16)],
                        acc_sh.at[pl.ds(16 * _RPT, 16)])

    pltpu.sync_copy(zerod_hbm, den_v)
    pltpu.sync_copy(att_hbm.at[cid], att_v)
    plsc.subcore_barrier()

    att_regs = [att_v[pl.ds(16 * j, 16)] for j in range(4)]
    lane = lax.iota(jnp.int32, 16)
    zero16 = jnp.zeros((16,), jnp.float32)

    def chunk_body(ci, carry):
        base = sid * _PW + ci * _E
        pltpu.sync_copy(src_hbm.at[pl.ds(base, _E)], src_v.at[pl.ds(0, _E)])
        pltpu.sync_copy(dst_hbm.at[pl.ds(base, _E)], dst_v.at[pl.ds(0, _E)])
        for e0 in range(0, _E, 16):
            sidx_v[0, pl.ds(e0, 16)] = lax.shift_right_logical(
                src_v[pl.ds(e0, 16)], 1)
            didx_v[0, pl.ds(e0, 16)] = lax.shift_right_logical(
                dst_v[pl.ds(e0, 16)], 1)
        c_pe = pltpu.async_copy(pe_hbm.at[cid].at[pl.ds(base, _E)], pe_v, sem0)
        c_hn = pltpu.async_copy(hn_hbm.at[cid].at[didx_v.at[0]], hn_v, sem1)
        c_g = pltpu.async_copy(g_hbm.at[cid].at[sidx_v.at[0]], g_v, sem2)
        c_pe.wait()
        c_hn.wait()
        c_g.wait()

        def edge_body(e, ecarry):
            sv = src_v[pl.ds(e, 16)]
            s0 = sv[0]
            even = (s0 & 1) == 0
            dv = dst_v[pl.ds(e, 16)]
            d_even = (dv[0] & 1) == 0
            xs = []
            ss = []
            for j in range(4):
                hj = jnp.where(d_even, hn_v[e, pl.ds(16 * j, 16)],
                               hn_v[e, pl.ds(_HH + 16 * j, 16)])
                gj = jnp.where(even, g_v[e, pl.ds(16 * j, 16)],
                               g_v[e, pl.ds(_HH + 16 * j, 16)])
                xj = pe_v[e, pl.ds(16 * j, 16)] + hj
                t = xj * gj
                t = jnp.maximum(t, 0.2 * t)
                ss.append(t * att_regs[j])
                xs.append(xj)
            tail = zero16
            msg = [None] * 4
            for hh in range(2):
                a = _allsum16(ss[2 * hh] + ss[2 * hh + 1])
                a = jnp.minimum(a, 60.0)
                wv = jnp.exp(a)
                msg[2 * hh] = xs[2 * hh] * wv
                msg[2 * hh + 1] = xs[2 * hh + 1] * wv
                tail = jnp.where(lane == hh, wv, tail)
            for j in range(4):
                mj = msg[j]
                msg_v[e, pl.ds(16 * j, 16)] = jnp.where(even, mj, zero16)
                msg_v[e, pl.ds(_HH + 16 * j, 16)] = jnp.where(even, zero16, mj)
            off = s0 * 2
            den_v[pl.ds(off, 16)] = den_v[pl.ds(off, 16)] + tail
            return ecarry

        lax.fori_loop(0, _E, edge_body, 0, unroll=4)
        pltpu.sync_copy(msg_v, acc_sh.at[sidx_v.at[0]], add=True)
        return carry

    lax.fori_loop(0, _NCHUNK, chunk_body, 0)
    plsc.subcore_barrier()
    pltpu.sync_copy(acc_sh.at[pl.ds(sid * _RPT, _RPT)],
                    acc_hbm.at[cid, pl.ds(sid * _RPT, _RPT)])

    @pl.when(sid == 0)
    def _out_tail():
        pltpu.sync_copy(acc_sh.at[pl.ds(16 * _RPT, 16)],
                        acc_hbm.at[cid, pl.ds(16 * _RPT, 16)])

    pltpu.sync_copy(den_v, den_hbm.at[cid, sid])


def _fin_body(n0_ref, n1_ref, d0_ref, d1_ref, bias_ref, lng_ref, lnb_ref, out_ref):
    num = jnp.concatenate([n0_ref[...], n1_ref[...]], axis=1)
    den0 = jnp.sum(d0_ref[...], axis=0)
    den1 = jnp.sum(d1_ref[...], axis=0)
    den = jnp.maximum(jnp.concatenate([den0, den1], axis=1), 1e-16)
    row = lax.broadcasted_iota(jnp.int32, (4, _HC), 0)
    col = lax.broadcasted_iota(jnp.int32, (4, _HC), 1)
    sel = jnp.where(row == col // _C, 1.0, 0.0).astype(jnp.float32)
    den_wide = jnp.dot(den, sel, preferred_element_type=jnp.float32)
    out = num / den_wide + bias_ref[...]
    mean = jnp.mean(out, axis=-1, keepdims=True)
    var = jnp.mean((out - mean) ** 2, axis=-1, keepdims=True)
    out_ref[...] = (out - mean) * lax.rsqrt(var + 1e-5) * lng_ref[...] + lnb_ref[...]


def _finalize(n0, n1, d0, d1, bias, ln_g, ln_b):
    return pl.pallas_call(
        _fin_body,
        grid=(_BS // _ROWS_C,),
        in_specs=[
            pl.BlockSpec((_ROWS_C, _HH), lambda i: (i, 0)),
            pl.BlockSpec((_ROWS_C, _HH), lambda i: (i, 0)),
            pl.BlockSpec((_NS, _ROWS_C, 2), lambda i: (0, i, 0)),
            pl.BlockSpec((_NS, _ROWS_C, 2), lambda i: (0, i, 0)),
            pl.BlockSpec((1, _HC), lambda i: (0, 0)),
            pl.BlockSpec((1, _HC), lambda i: (0, 0)),
            pl.BlockSpec((1, _HC), lambda i: (0, 0)),
        ],
        out_specs=pl.BlockSpec((_ROWS_C, _HC), lambda i: (i, 0)),
        out_shape=jax.ShapeDtypeStruct((_BS, _HC), jnp.float32),
    )(n0, n1, d0, d1,
      bias.reshape(1, _HC), ln_g.reshape(1, _HC), ln_b.reshape(1, _HC))


def kernel(edge_index, edge_x, node_x, pe_enc, W_l, b_l, W_r, b_r, att, bias, ln_g, ln_b):
    src = edge_index[0]
    dst = edge_index[1]
    W_r1 = W_r[:_DIM]
    W_r2 = W_r[_DIM:]
    pe_pair = _pe_proj(pe_enc, W_r2, b_r)
    g_pair, hn_pair = _precompute(edge_x, node_x, W_l, W_r1, b_l)
    g_pack = g_pair.reshape(2, _BS // 2, _HC)
    hn_pack = hn_pair.reshape(2, _NV // 2, _HC)
    att_pair = att.reshape(2, _HH)
    zeros = jnp.zeros((_RPT, _HC), jnp.float32)
    zerod = jnp.zeros((_DROW,), jnp.float32)
    acc, den = _sc_edges(src, dst, pe_pair, hn_pack, g_pack, att_pair,
                         zeros, zerod)
    nums = acc[:, : _BS // 2, :].reshape(2, _BS, _HH)
    dens = den[:, :, : 2 * _BS].reshape(2, _NS, _BS, 2)
    return _finalize(nums[0], nums[1], dens[0], dens[1], bias, ln_g, ln_b)


# speedup vs baseline: 1.2202x; 1.2202x over previous
import functools

import jax
import jax.numpy as jnp
from jax import lax
from jax.experimental import pallas as pl
from jax.experimental.pallas import tpu as pltpu
from jax.experimental.pallas import tpu_sc as plsc

_P = 320000
_BS = 10000
_NV = 10000
_DIM = 128
_H = 4
_C = 32
_HC = _H * _C
_HH = 64

_NS = 16
_PW = _P // _NS
_E = 80
_NCHUNK = _PW // _E

_ACC_R = 5008
_RPT = 312
_DROW = 2 * _BS + 96

_ROWS_A = 1000
_ROWS_C = 1000


def _peproj_body(pe_ref, w_ref, b_ref, out_ref):
    full = (
        jnp.dot(pe_ref[...], w_ref[...], preferred_element_type=jnp.float32)
        + b_ref[...]
    )
    out_ref[0] = full[:, :_HH]
    out_ref[1] = full[:, _HH:]


def _pe_proj(pe_enc, W_r2, b_r):
    return pl.pallas_call(
        _peproj_body,
        grid=(_P // _ROWS_A,),
        in_specs=[
            pl.BlockSpec((_ROWS_A, 2 * _DIM), lambda i: (i, 0)),
            pl.BlockSpec((2 * _DIM, _HC), lambda i: (0, 0)),
            pl.BlockSpec((1, _HC), lambda i: (0, 0)),
        ],
        out_specs=pl.BlockSpec((2, _ROWS_A, _HH), lambda i: (0, i, 0)),
        out_shape=jax.ShapeDtypeStruct((2, _P, _HH), jnp.float32),
    )(pe_enc, W_r2, b_r.reshape(1, _HC))


def _pre_body(ex_ref, nx_ref, wl_ref, wr1_ref, bl_ref, g_ref, hn_ref):
    esum = ex_ref[:, :_DIM] + ex_ref[:, _DIM:]
    gfull = (
        jnp.dot(esum, wl_ref[...], preferred_element_type=jnp.float32)
        + 2.0 * bl_ref[...]
    )
    g_ref[0] = gfull[:, :_HH]
    g_ref[1] = gfull[:, _HH:]
    hfull = jnp.dot(nx_ref[...], wr1_ref[...], preferred_element_type=jnp.float32)
    hn_ref[0] = hfull[:, :_HH]
    hn_ref[1] = hfull[:, _HH:]


def _precompute(edge_x, node_x, W_l, W_r1, b_l):
    return pl.pallas_call(
        _pre_body,
        out_shape=(
            jax.ShapeDtypeStruct((2, _BS, _HH), jnp.float32),
            jax.ShapeDtypeStruct((2, _NV, _HH), jnp.float32),
        ),
    )(edge_x, node_x, W_l, W_r1, b_l.reshape(1, _HC))


_sc_mesh = plsc.VectorSubcoreMesh(core_axis_name="c", subcore_axis_name="s")

_GDN = lax.GatherDimensionNumbers(
    offset_dims=(), collapsed_slice_dims=(0,), start_index_map=(0,))


def _allsum16(v):
    for s in (8, 4, 2, 1):
        idx = (lax.iota(jnp.int32, 16) ^ s).reshape(16, 1)
        v = v + lax.gather(v, idx, _GDN, slice_sizes=(1,),
                           mode=lax.GatherScatterMode.PROMISE_IN_BOUNDS)
    return v


@functools.partial(
    pl.kernel,
    out_type=(
        jax.ShapeDtypeStruct((2, _ACC_R, _HC), jnp.float32),
        jax.ShapeDtypeStruct((2, _NS, _DROW), jnp.float32),
    ),
    mesh=_sc_mesh,
    scratch_types=[
        pltpu.VMEM((2, _E + 16), jnp.int32),
        pltpu.VMEM((2, _E + 16), jnp.int32),
        pltpu.VMEM((2, 2, _E), jnp.int32),
        pltpu.VMEM((2, _E, _HH), jnp.float32),
        pltpu.VMEM((2, _E, _HC), jnp.float32),
        pltpu.VMEM((2, _E, _HC), jnp.float32),
        pltpu.VMEM((_DROW,), jnp.float32),
        pltpu.VMEM((_HH,), jnp.float32),
        pltpu.VMEM_SHARED((_ACC_R, _HC), jnp.float32),
        pltpu.SemaphoreType.DMA,
        pltpu.SemaphoreType.DMA,
    ],
)
def _sc_edges(src_hbm, dst_hbm, pe_hbm, hn_hbm, g_hbm, att_hbm, zero_hbm, zerod_hbm,
              acc_hbm, den_hbm,
              idx_v, dst_v, sidx_v, pe_v, hn_v, g_v, den_v, att_v,
              acc_sh, sem0, sem1):
    cid = lax.axis_index("c")
    sid = lax.axis_index("s")
    sems = (sem0, sem1)

    pltpu.sync_copy(zero_hbm, acc_sh.at[pl.ds(sid * _RPT, _RPT)])

    @pl.when(sid == 0)
    def _init_tail():
        pltpu.sync_copy(zero_hbm.at[pl.ds(0, 16)],
                        acc_sh.at[pl.ds(16 * _RPT, 16)])

    pltpu.sync_copy(zerod_hbm, den_v)
    pltpu.sync_copy(att_hbm.at[cid], att_v)
    plsc.subcore_barrier()

    att_regs = [att_v[pl.ds(16 * j, 16)] for j in range(4)]
    lane = lax.iota(jnp.int32, 16)
    zero16 = jnp.zeros((16,), jnp.float32)

    def issue(ci, b):
        base = sid * _PW + ci * _E
        pltpu.sync_copy(src_hbm.at[pl.ds(base, _E)], idx_v.at[b, pl.ds(0, _E)])
        pltpu.sync_copy(dst_hbm.at[pl.ds(base, _E)], dst_v.at[b, pl.ds(0, _E)])
        for e0 in range(0, _E, 16):
            sidx_v[b, 0, pl.ds(e0, 16)] = lax.shift_right_logical(
                idx_v[b, pl.ds(e0, 16)], 1)
            sidx_v[b, 1, pl.ds(e0, 16)] = lax.shift_right_logical(
                dst_v[b, pl.ds(e0, 16)], 1)
        pltpu.async_copy(pe_hbm.at[cid].at[pl.ds(base, _E)], pe_v.at[b], sems[b])
        pltpu.async_copy(hn_hbm.at[cid].at[sidx_v.at[b, 1]], hn_v.at[b], sems[b])
        pltpu.async_copy(g_hbm.at[cid].at[sidx_v.at[b, 0]], g_v.at[b], sems[b])

    def work(ci, b):
        pltpu.make_async_copy(
            pe_hbm.at[cid].at[pl.ds(0, _E)], pe_v.at[b], sems[b]).wait()
        pltpu.make_async_copy(
            hn_hbm.at[cid].at[pl.ds(0, _E)], hn_v.at[b], sems[b]).wait()
        pltpu.make_async_copy(
            g_hbm.at[cid].at[pl.ds(0, _E)], g_v.at[b], sems[b]).wait()

        def edge_body(e, ecarry):
            sv = idx_v[b, pl.ds(e, 16)]
            s0 = sv[0]
            even = (s0 & 1) == 0
            dv = dst_v[b, pl.ds(e, 16)]
            d_even = (dv[0] & 1) == 0
            xs = []
            ss = []
            for j in range(4):
                hj = jnp.where(d_even, hn_v[b, e, pl.ds(16 * j, 16)],
                               hn_v[b, e, pl.ds(_HH + 16 * j, 16)])
                gj = jnp.where(even, g_v[b, e, pl.ds(16 * j, 16)],
                               g_v[b, e, pl.ds(_HH + 16 * j, 16)])
                xj = pe_v[b, e, pl.ds(16 * j, 16)] + hj
                t = xj * gj
                t = jnp.maximum(t, 0.2 * t)
                ss.append(t * att_regs[j])
                xs.append(xj)
            tail = zero16
            msg = [None] * 4
            for hh in range(2):
                a = _allsum16(ss[2 * hh] + ss[2 * hh + 1])
                a = jnp.minimum(a, 60.0)
                wv = jnp.exp(a)
                msg[2 * hh] = xs[2 * hh] * wv
                msg[2 * hh + 1] = xs[2 * hh + 1] * wv
                tail = jnp.where(lane == hh, wv, tail)
            for j in range(4):
                mj = msg[j]
                g_v[b, e, pl.ds(16 * j, 16)] = jnp.where(even, mj, zero16)
                g_v[b, e, pl.ds(_HH + 16 * j, 16)] = jnp.where(even, zero16, mj)
            off = s0 * 2
            den_v[pl.ds(off, 16)] = den_v[pl.ds(off, 16)] + tail
            return ecarry

        lax.fori_loop(0, _E, edge_body, 0)
        pltpu.sync_copy(g_v.at[b], acc_sh.at[sidx_v.at[b, 0]], add=True)

    issue(0, 0)

    def chunk_pair(i, carry):
        ci = i * 2
        issue(ci + 1, 1)
        work(ci, 0)

        @pl.when(ci + 2 < _NCHUNK)
        def _prefetch():
            issue(ci + 2, 0)

        work(ci + 1, 1)
        return carry

    lax.fori_loop(0, _NCHUNK // 2, chunk_pair, 0)
    plsc.subcore_barrier()
    pltpu.sync_copy(acc_sh.at[pl.ds(sid * _RPT, _RPT)],
                    acc_hbm.at[cid, pl.ds(sid * _RPT, _RPT)])

    @pl.when(sid == 0)
    def _out_tail():
        pltpu.sync_copy(acc_sh.at[pl.ds(16 * _RPT, 16)],
                        acc_hbm.at[cid, pl.ds(16 * _RPT, 16)])

    pltpu.sync_copy(den_v, den_hbm.at[cid, sid])


def _fin_body(n0_ref, n1_ref, d0_ref, d1_ref, bias_ref, lng_ref, lnb_ref, out_ref):
    num = jnp.concatenate([n0_ref[...], n1_ref[...]], axis=1)
    den0 = jnp.sum(d0_ref[...], axis=0)
    den1 = jnp.sum(d1_ref[...], axis=0)
    den = jnp.maximum(jnp.concatenate([den0, den1], axis=1), 1e-16)
    row = lax.broadcasted_iota(jnp.int32, (4, _HC), 0)
    col = lax.broadcasted_iota(jnp.int32, (4, _HC), 1)
    sel = jnp.where(row == col // _C, 1.0, 0.0).astype(jnp.float32)
    den_wide = jnp.dot(den, sel, preferred_element_type=jnp.float32)
    out = num / den_wide + bias_ref[...]
    mean = jnp.mean(out, axis=-1, keepdims=True)
    var = jnp.mean((out - mean) ** 2, axis=-1, keepdims=True)
    out_ref[...] = (out - mean) * lax.rsqrt(var + 1e-5) * lng_ref[...] + lnb_ref[...]


def _finalize(n0, n1, d0, d1, bias, ln_g, ln_b):
    return pl.pallas_call(
        _fin_body,
        grid=(_BS // _ROWS_C,),
        in_specs=[
            pl.BlockSpec((_ROWS_C, _HH), lambda i: (i, 0)),
            pl.BlockSpec((_ROWS_C, _HH), lambda i: (i, 0)),
            pl.BlockSpec((_NS, _ROWS_C, 2), lambda i: (0, i, 0)),
            pl.BlockSpec((_NS, _ROWS_C, 2), lambda i: (0, i, 0)),
            pl.BlockSpec((1, _HC), lambda i: (0, 0)),
            pl.BlockSpec((1, _HC), lambda i: (0, 0)),
            pl.BlockSpec((1, _HC), lambda i: (0, 0)),
        ],
        out_specs=pl.BlockSpec((_ROWS_C, _HC), lambda i: (i, 0)),
        out_shape=jax.ShapeDtypeStruct((_BS, _HC), jnp.float32),
    )(n0, n1, d0, d1,
      bias.reshape(1, _HC), ln_g.reshape(1, _HC), ln_b.reshape(1, _HC))


def kernel(edge_index, edge_x, node_x, pe_enc, W_l, b_l, W_r, b_r, att, bias, ln_g, ln_b):
    src = edge_index[0]
    dst = edge_index[1]
    W_r1 = W_r[:_DIM]
    W_r2 = W_r[_DIM:]
    pe_pair = _pe_proj(pe_enc, W_r2, b_r)
    g_pair, hn_pair = _precompute(edge_x, node_x, W_l, W_r1, b_l)
    g_pack = g_pair.reshape(2, _BS // 2, _HC)
    hn_pack = hn_pair.reshape(2, _NV // 2, _HC)
    att_pair = att.reshape(2, _HH)
    zeros = jnp.zeros((_RPT, _HC), jnp.float32)
    zerod = jnp.zeros((_DROW,), jnp.float32)
    acc, den = _sc_edges(src, dst, pe_pair, hn_pack, g_pack, att_pair,
                         zeros, zerod)
    nums = acc[:, : _BS // 2, :].reshape(2, _BS, _HH)
    dens = den[:, :, : 2 * _BS].reshape(2, _NS, _BS, 2)
    return _finalize(nums[0], nums[1], dens[0], dens[1], bias, ln_g, ln_b)


# speedup vs baseline: 1.2214x; 1.0010x over previous
import functools

import jax
import jax.numpy as jnp
from jax import lax
from jax.experimental import pallas as pl
from jax.experimental.pallas import tpu as pltpu
from jax.experimental.pallas import tpu_sc as plsc

_P = 320000
_BS = 10000
_NV = 10000
_DIM = 128
_H = 4
_C = 32
_HC = _H * _C
_HH = 64

_NS = 16
_PW = _P // _NS
_E = 80
_NCHUNK = _PW // _E

_ACC_R = 5008
_RPT = 312
_DROW = 2 * _BS + 96

_ROWS_A = 1000
_ROWS_C = 1000


def _peproj_body(pe_ref, w_ref, b_ref, out_ref):
    full = (
        jnp.dot(pe_ref[...], w_ref[...], preferred_element_type=jnp.float32)
        + b_ref[...]
    )
    out_ref[0] = full[:, :_HH]
    out_ref[1] = full[:, _HH:]


def _pe_proj(pe_enc, W_r2, b_r):
    return pl.pallas_call(
        _peproj_body,
        grid=(_P // _ROWS_A,),
        in_specs=[
            pl.BlockSpec((_ROWS_A, 2 * _DIM), lambda i: (i, 0)),
            pl.BlockSpec((2 * _DIM, _HC), lambda i: (0, 0)),
            pl.BlockSpec((1, _HC), lambda i: (0, 0)),
        ],
        out_specs=pl.BlockSpec((2, _ROWS_A, _HH), lambda i: (0, i, 0)),
        out_shape=jax.ShapeDtypeStruct((2, _P, _HH), jnp.float32),
    )(pe_enc, W_r2, b_r.reshape(1, _HC))


def _pre_body(ex_ref, nx_ref, wl_ref, wr1_ref, bl_ref, g_ref, hn_ref):
    esum = ex_ref[:, :_DIM] + ex_ref[:, _DIM:]
    gfull = (
        jnp.dot(esum, wl_ref[...], preferred_element_type=jnp.float32)
        + 2.0 * bl_ref[...]
    )
    g_ref[0] = gfull[:, :_HH]
    g_ref[1] = gfull[:, _HH:]
    hfull = jnp.dot(nx_ref[...], wr1_ref[...], preferred_element_type=jnp.float32)
    hn_ref[0] = hfull[:, :_HH]
    hn_ref[1] = hfull[:, _HH:]


def _precompute(edge_x, node_x, W_l, W_r1, b_l):
    return pl.pallas_call(
        _pre_body,
        out_shape=(
            jax.ShapeDtypeStruct((2, _BS, _HH), jnp.float32),
            jax.ShapeDtypeStruct((2, _NV, _HH), jnp.float32),
        ),
    )(edge_x, node_x, W_l, W_r1, b_l.reshape(1, _HC))


_sc_mesh = plsc.VectorSubcoreMesh(core_axis_name="c", subcore_axis_name="s")

_GDN = lax.GatherDimensionNumbers(
    offset_dims=(), collapsed_slice_dims=(0,), start_index_map=(0,))


def _allsum16(v):
    for s in (8, 4, 2, 1):
        idx = (lax.iota(jnp.int32, 16) ^ s).reshape(16, 1)
        v = v + lax.gather(v, idx, _GDN, slice_sizes=(1,),
                           mode=lax.GatherScatterMode.PROMISE_IN_BOUNDS)
    return v


@functools.partial(
    pl.kernel,
    out_type=(
        jax.ShapeDtypeStruct((2, _ACC_R, _HC), jnp.float32),
        jax.ShapeDtypeStruct((2, _NS, _DROW), jnp.float32),
    ),
    mesh=_sc_mesh,
    scratch_types=[
        pltpu.VMEM((2, _E + 16), jnp.int32),
        pltpu.VMEM((2, _E + 16), jnp.int32),
        pltpu.VMEM((2, 2, _E), jnp.int32),
        pltpu.VMEM((2, _E, _HH), jnp.float32),
        pltpu.VMEM((2, _E, _HC), jnp.float32),
        pltpu.VMEM((2, _E, _HC), jnp.float32),
        pltpu.VMEM((_DROW,), jnp.float32),
        pltpu.VMEM((_HH,), jnp.float32),
        pltpu.VMEM_SHARED((_ACC_R, _HC), jnp.float32),
        pltpu.SemaphoreType.DMA,
        pltpu.SemaphoreType.DMA,
    ],
)
def _sc_edges(src_hbm, dst_hbm, pe_hbm, hn_hbm, g_hbm, att_hbm, zero_hbm, zerod_hbm,
              acc_hbm, den_hbm,
              idx_v, dst_v, sidx_v, pe_v, hn_v, g_v, den_v, att_v,
              acc_sh, sem0, sem1):
    cid = lax.axis_index("c")
    sid = lax.axis_index("s")
    sems = (sem0, sem1)

    pltpu.sync_copy(zero_hbm, acc_sh.at[pl.ds(sid * _RPT, _RPT)])

    @pl.when(sid == 0)
    def _init_tail():
        pltpu.sync_copy(zero_hbm.at[pl.ds(0, 16)],
                        acc_sh.at[pl.ds(16 * _RPT, 16)])

    pltpu.sync_copy(zerod_hbm, den_v)
    pltpu.sync_copy(att_hbm.at[cid], att_v)
    plsc.subcore_barrier()

    att_regs = [att_v[pl.ds(16 * j, 16)] for j in range(4)]
    lane = lax.iota(jnp.int32, 16)
    zero16 = jnp.zeros((16,), jnp.float32)

    def issue(ci, b):
        base = sid * _PW + ci * _E
        pltpu.sync_copy(src_hbm.at[pl.ds(base, _E)], idx_v.at[b, pl.ds(0, _E)])
        pltpu.sync_copy(dst_hbm.at[pl.ds(base, _E)], dst_v.at[b, pl.ds(0, _E)])
        for e0 in range(0, _E, 16):
            sidx_v[b, 0, pl.ds(e0, 16)] = lax.shift_right_logical(
                idx_v[b, pl.ds(e0, 16)], 1)
            sidx_v[b, 1, pl.ds(e0, 16)] = lax.shift_right_logical(
                dst_v[b, pl.ds(e0, 16)], 1)
        pltpu.async_copy(pe_hbm.at[cid].at[pl.ds(base, _E)], pe_v.at[b], sems[b])
        pltpu.async_copy(hn_hbm.at[cid].at[sidx_v.at[b, 1]], hn_v.at[b], sems[b])
        pltpu.async_copy(g_hbm.at[cid].at[sidx_v.at[b, 0]], g_v.at[b], sems[b])

    def work(ci, b):
        pltpu.make_async_copy(
            pe_hbm.at[cid].at[pl.ds(0, _E)], pe_v.at[b], sems[b]).wait()
        pltpu.make_async_copy(
            hn_hbm.at[cid].at[pl.ds(0, _E)], hn_v.at[b], sems[b]).wait()
        pltpu.make_async_copy(
            g_hbm.at[cid].at[pl.ds(0, _E)], g_v.at[b], sems[b]).wait()

        def edge_body(e, ecarry):
            sv = idx_v[b, pl.ds(e, 16)]
            s0 = sv[0]
            even = (s0 & 1) == 0
            dv = dst_v[b, pl.ds(e, 16)]
            d_even = (dv[0] & 1) == 0
            xs = []
            ss = []
            for j in range(4):
                hj = jnp.where(d_even, hn_v[b, e, pl.ds(16 * j, 16)],
                               hn_v[b, e, pl.ds(_HH + 16 * j, 16)])
                gj = jnp.where(even, g_v[b, e, pl.ds(16 * j, 16)],
                               g_v[b, e, pl.ds(_HH + 16 * j, 16)])
                xj = pe_v[b, e, pl.ds(16 * j, 16)] + hj
                t = xj * gj
                t = jnp.maximum(t, 0.2 * t)
                ss.append(t * att_regs[j])
                xs.append(xj)
            tail = zero16
            msg = [None] * 4
            for hh in range(2):
                a = _allsum16(ss[2 * hh] + ss[2 * hh + 1])
                a = jnp.minimum(a, 60.0)
                wv = jnp.exp(a)
                msg[2 * hh] = xs[2 * hh] * wv
                msg[2 * hh + 1] = xs[2 * hh + 1] * wv
                tail = jnp.where(lane == hh, wv, tail)
            for j in range(4):
                mj = msg[j]
                g_v[b, e, pl.ds(16 * j, 16)] = jnp.where(even, mj, zero16)
                g_v[b, e, pl.ds(_HH + 16 * j, 16)] = jnp.where(even, zero16, mj)
            off = s0 * 2
            den_v[pl.ds(off, 16)] = den_v[pl.ds(off, 16)] + tail
            return ecarry

        lax.fori_loop(0, _E, edge_body, 0, unroll=2)
        pltpu.sync_copy(g_v.at[b], acc_sh.at[sidx_v.at[b, 0]], add=True)

    issue(0, 0)

    def chunk_pair(i, carry):
        ci = i * 2
        issue(ci + 1, 1)
        work(ci, 0)

        @pl.when(ci + 2 < _NCHUNK)
        def _prefetch():
            issue(ci + 2, 0)

        work(ci + 1, 1)
        return carry

    lax.fori_loop(0, _NCHUNK // 2, chunk_pair, 0)
    plsc.subcore_barrier()
    pltpu.sync_copy(acc_sh.at[pl.ds(sid * _RPT, _RPT)],
                    acc_hbm.at[cid, pl.ds(sid * _RPT, _RPT)])

    @pl.when(sid == 0)
    def _out_tail():
        pltpu.sync_copy(acc_sh.at[pl.ds(16 * _RPT, 16)],
                        acc_hbm.at[cid, pl.ds(16 * _RPT, 16)])

    pltpu.sync_copy(den_v, den_hbm.at[cid, sid])


def _fin_body(n0_ref, n1_ref, d0_ref, d1_ref, bias_ref, lng_ref, lnb_ref, out_ref):
    num = jnp.concatenate([n0_ref[...], n1_ref[...]], axis=1)
    den0 = jnp.sum(d0_ref[...], axis=0)
    den1 = jnp.sum(d1_ref[...], axis=0)
    den = jnp.maximum(jnp.concatenate([den0, den1], axis=1), 1e-16)
    row = lax.broadcasted_iota(jnp.int32, (4, _HC), 0)
    col = lax.broadcasted_iota(jnp.int32, (4, _HC), 1)
    sel = jnp.where(row == col // _C, 1.0, 0.0).astype(jnp.float32)
    den_wide = jnp.dot(den, sel, preferred_element_type=jnp.float32)
    out = num / den_wide + bias_ref[...]
    mean = jnp.mean(out, axis=-1, keepdims=True)
    var = jnp.mean((out - mean) ** 2, axis=-1, keepdims=True)
    out_ref[...] = (out - mean) * lax.rsqrt(var + 1e-5) * lng_ref[...] + lnb_ref[...]


def _finalize(n0, n1, d0, d1, bias, ln_g, ln_b):
    return pl.pallas_call(
        _fin_body,
        grid=(_BS // _ROWS_C,),
        in_specs=[
            pl.BlockSpec((_ROWS_C, _HH), lambda i: (i, 0)),
            pl.BlockSpec((_ROWS_C, _HH), lambda i: (i, 0)),
            pl.BlockSpec((_NS, _ROWS_C, 2), lambda i: (0, i, 0)),
            pl.BlockSpec((_NS, _ROWS_C, 2), lambda i: (0, i, 0)),
            pl.BlockSpec((1, _HC), lambda i: (0, 0)),
            pl.BlockSpec((1, _HC), lambda i: (0, 0)),
            pl.BlockSpec((1, _HC), lambda i: (0, 0)),
        ],
        out_specs=pl.BlockSpec((_ROWS_C, _HC), lambda i: (i, 0)),
        out_shape=jax.ShapeDtypeStruct((_BS, _HC), jnp.float32),
    )(n0, n1, d0, d1,
      bias.reshape(1, _HC), ln_g.reshape(1, _HC), ln_b.reshape(1, _HC))


def kernel(edge_index, edge_x, node_x, pe_enc, W_l, b_l, W_r, b_r, att, bias, ln_g, ln_b):
    src = edge_index[0]
    dst = edge_index[1]
    W_r1 = W_r[:_DIM]
    W_r2 = W_r[_DIM:]
    pe_pair = _pe_proj(pe_enc, W_r2, b_r)
    g_pair, hn_pair = _precompute(edge_x, node_x, W_l, W_r1, b_l)
    g_pack = g_pair.reshape(2, _BS // 2, _HC)
    hn_pack = hn_pair.reshape(2, _NV // 2, _HC)
    att_pair = att.reshape(2, _HH)
    zeros = jnp.zeros((_RPT, _HC), jnp.float32)
    zerod = jnp.zeros((_DROW,), jnp.float32)
    acc, den = _sc_edges(src, dst, pe_pair, hn_pack, g_pack, att_pair,
                         zeros, zerod)
    nums = acc[:, : _BS // 2, :].reshape(2, _BS, _HH)
    dens = den[:, :, : 2 * _BS].reshape(2, _NS, _BS, 2)
    return _finalize(nums[0], nums[1], dens[0], dens[1], bias, ln_g, ln_b)


# speedup vs baseline: 1.3820x; 1.1315x over previous
import functools

import jax
import jax.numpy as jnp
from jax import lax
from jax.experimental import pallas as pl
from jax.experimental.pallas import tpu as pltpu
from jax.experimental.pallas import tpu_sc as plsc

_P = 320000
_BS = 10000
_NV = 10000
_DIM = 128
_H = 4
_C = 32
_HC = _H * _C
_HH = 64

_NS = 16
_PW = _P // _NS
_E = 80
_NCHUNK = _PW // _E

_ACC_R = 5008
_RPT = 312
_DROW = 2 * _BS + 96

_ROWS_A = 1000
_ROWS_C = 1000


def _peproj_body(pe_ref, w_ref, b_ref, out_ref):
    full = (
        jnp.dot(pe_ref[...], w_ref[...], preferred_element_type=jnp.float32)
        + b_ref[...]
    )
    out_ref[0] = full[:, :_HH]
    out_ref[1] = full[:, _HH:]


def _pe_proj(pe_enc, W_r2, b_r):
    return pl.pallas_call(
        _peproj_body,
        grid=(_P // _ROWS_A,),
        in_specs=[
            pl.BlockSpec((_ROWS_A, 2 * _DIM), lambda i: (i, 0)),
            pl.BlockSpec((2 * _DIM, _HC), lambda i: (0, 0)),
            pl.BlockSpec((1, _HC), lambda i: (0, 0)),
        ],
        out_specs=pl.BlockSpec((2, _ROWS_A, _HH), lambda i: (0, i, 0)),
        out_shape=jax.ShapeDtypeStruct((2, _P, _HH), jnp.float32),
    )(pe_enc, W_r2, b_r.reshape(1, _HC))


def _pre_body(ex_ref, nx_ref, wl_ref, wr1_ref, bl_ref, g_ref, hn_ref):
    esum = ex_ref[:, :_DIM] + ex_ref[:, _DIM:]
    gfull = (
        jnp.dot(esum, wl_ref[...], preferred_element_type=jnp.float32)
        + 2.0 * bl_ref[...]
    )
    g_ref[0] = jnp.concatenate([gfull[:, :_HH], gfull[:, :_HH]], axis=1)
    g_ref[1] = jnp.concatenate([gfull[:, _HH:], gfull[:, _HH:]], axis=1)
    hfull = jnp.dot(nx_ref[...], wr1_ref[...], preferred_element_type=jnp.float32)
    hn_ref[0] = jnp.concatenate([hfull[:, :_HH], hfull[:, :_HH]], axis=1)
    hn_ref[1] = jnp.concatenate([hfull[:, _HH:], hfull[:, _HH:]], axis=1)


def _precompute(edge_x, node_x, W_l, W_r1, b_l):
    return pl.pallas_call(
        _pre_body,
        out_shape=(
            jax.ShapeDtypeStruct((2, _BS, _HC), jnp.float32),
            jax.ShapeDtypeStruct((2, _NV, _HC), jnp.float32),
        ),
    )(edge_x, node_x, W_l, W_r1, b_l.reshape(1, _HC))


_sc_mesh = plsc.VectorSubcoreMesh(core_axis_name="c", subcore_axis_name="s")

_GDN = lax.GatherDimensionNumbers(
    offset_dims=(), collapsed_slice_dims=(0,), start_index_map=(0,))


def _allsum16(v):
    for s in (8, 4, 2, 1):
        idx = (lax.iota(jnp.int32, 16) ^ s).reshape(16, 1)
        v = v + lax.gather(v, idx, _GDN, slice_sizes=(1,),
                           mode=lax.GatherScatterMode.PROMISE_IN_BOUNDS)
    return v


@functools.partial(
    pl.kernel,
    out_type=(
        jax.ShapeDtypeStruct((2, _ACC_R, _HC), jnp.float32),
        jax.ShapeDtypeStruct((2, _NS, _DROW), jnp.float32),
    ),
    mesh=_sc_mesh,
    scratch_types=[
        pltpu.VMEM((2, _E + 16), jnp.int32),
        pltpu.VMEM((2, _E + 16), jnp.int32),
        pltpu.VMEM((2, 1, _E), jnp.int32),
        pltpu.VMEM((2, _E, _HH), jnp.float32),
        pltpu.VMEM((2, _E, _HC), jnp.float32),
        pltpu.VMEM((2, _E, _HC), jnp.float32),
        pltpu.VMEM((_DROW,), jnp.float32),
        pltpu.VMEM((_HH,), jnp.float32),
        pltpu.VMEM_SHARED((_ACC_R, _HC), jnp.float32),
        pltpu.SemaphoreType.DMA,
        pltpu.SemaphoreType.DMA,
    ],
)
def _sc_edges(src_hbm, dst_hbm, pe_hbm, hn_hbm, g_hbm, att_hbm, zero_hbm, zerod_hbm,
              acc_hbm, den_hbm,
              idx_v, dst_v, sidx_v, pe_v, hn_v, g_v, den_v, att_v,
              acc_sh, sem0, sem1):
    cid = lax.axis_index("c")
    sid = lax.axis_index("s")
    sems = (sem0, sem1)

    pltpu.sync_copy(zero_hbm, acc_sh.at[pl.ds(sid * _RPT, _RPT)])

    @pl.when(sid == 0)
    def _init_tail():
        pltpu.sync_copy(zero_hbm.at[pl.ds(0, 16)],
                        acc_sh.at[pl.ds(16 * _RPT, 16)])

    pltpu.sync_copy(zerod_hbm, den_v)
    pltpu.sync_copy(att_hbm.at[cid], att_v)
    plsc.subcore_barrier()

    att_regs = [att_v[pl.ds(16 * j, 16)] for j in range(4)]
    lane = lax.iota(jnp.int32, 16)
    zero16 = jnp.zeros((16,), jnp.float32)

    def issue(ci, b):
        base = sid * _PW + ci * _E
        pltpu.sync_copy(src_hbm.at[pl.ds(base, _E)], idx_v.at[b, pl.ds(0, _E)])
        pltpu.sync_copy(dst_hbm.at[pl.ds(base, _E)], dst_v.at[b, pl.ds(0, _E)])
        for e0 in range(0, _E, 16):
            sidx_v[b, 0, pl.ds(e0, 16)] = lax.shift_right_logical(
                idx_v[b, pl.ds(e0, 16)], 1)
        pltpu.async_copy(pe_hbm.at[cid].at[pl.ds(base, _E)], pe_v.at[b], sems[b])
        pltpu.async_copy(hn_hbm.at[cid].at[dst_v.at[b, pl.ds(0, _E)]],
                         hn_v.at[b], sems[b])
        pltpu.async_copy(g_hbm.at[cid].at[idx_v.at[b, pl.ds(0, _E)]],
                         g_v.at[b], sems[b])

    def work(ci, b):
        pltpu.make_async_copy(
            pe_hbm.at[cid].at[pl.ds(0, _E)], pe_v.at[b], sems[b]).wait()
        pltpu.make_async_copy(
            hn_hbm.at[cid].at[pl.ds(0, _E)], hn_v.at[b], sems[b]).wait()
        pltpu.make_async_copy(
            g_hbm.at[cid].at[pl.ds(0, _E)], g_v.at[b], sems[b]).wait()

        def edge_body(e, ecarry):
            sv = idx_v[b, pl.ds(e, 16)]
            s0 = sv[0]
            even = (s0 & 1) == 0
            xs = []
            ss = []
            for j in range(4):
                xj = pe_v[b, e, pl.ds(16 * j, 16)] + hn_v[b, e, pl.ds(16 * j, 16)]
                t = xj * g_v[b, e, pl.ds(16 * j, 16)]
                t = jnp.maximum(t, 0.2 * t)
                ss.append(t * att_regs[j])
                xs.append(xj)
            tail = zero16
            msg = [None] * 4
            for hh in range(2):
                a = _allsum16(ss[2 * hh] + ss[2 * hh + 1])
                a = jnp.minimum(a, 60.0)
                wv = jnp.exp(a)
                msg[2 * hh] = xs[2 * hh] * wv
                msg[2 * hh + 1] = xs[2 * hh + 1] * wv
                tail = jnp.where(lane == hh, wv, tail)
            for j in range(4):
                mj = msg[j]
                g_v[b, e, pl.ds(16 * j, 16)] = jnp.where(even, mj, zero16)
                g_v[b, e, pl.ds(_HH + 16 * j, 16)] = jnp.where(even, zero16, mj)
            off = s0 * 2
            den_v[pl.ds(off, 16)] = den_v[pl.ds(off, 16)] + tail
            return ecarry

        lax.fori_loop(0, _E, edge_body, 0, unroll=2)
        pltpu.sync_copy(g_v.at[b], acc_sh.at[sidx_v.at[b, 0]], add=True)

    issue(0, 0)

    def chunk_pair(i, carry):
        ci = i * 2
        issue(ci + 1, 1)
        work(ci, 0)

        @pl.when(ci + 2 < _NCHUNK)
        def _prefetch():
            issue(ci + 2, 0)

        work(ci + 1, 1)
        return carry

    lax.fori_loop(0, _NCHUNK // 2, chunk_pair, 0)
    plsc.subcore_barrier()
    pltpu.sync_copy(acc_sh.at[pl.ds(sid * _RPT, _RPT)],
                    acc_hbm.at[cid, pl.ds(sid * _RPT, _RPT)])

    @pl.when(sid == 0)
    def _out_tail():
        pltpu.sync_copy(acc_sh.at[pl.ds(16 * _RPT, 16)],
                        acc_hbm.at[cid, pl.ds(16 * _RPT, 16)])

    pltpu.sync_copy(den_v, den_hbm.at[cid, sid])


def _fin_body(n0_ref, n1_ref, d0_ref, d1_ref, bias_ref, lng_ref, lnb_ref, out_ref):
    num = jnp.concatenate([n0_ref[...], n1_ref[...]], axis=1)
    den0 = jnp.sum(d0_ref[...], axis=0)
    den1 = jnp.sum(d1_ref[...], axis=0)
    den = jnp.maximum(jnp.concatenate([den0, den1], axis=1), 1e-16)
    row = lax.broadcasted_iota(jnp.int32, (4, _HC), 0)
    col = lax.broadcasted_iota(jnp.int32, (4, _HC), 1)
    sel = jnp.where(row == col // _C, 1.0, 0.0).astype(jnp.float32)
    den_wide = jnp.dot(den, sel, preferred_element_type=jnp.float32)
    out = num / den_wide + bias_ref[...]
    mean = jnp.mean(out, axis=-1, keepdims=True)
    var = jnp.mean((out - mean) ** 2, axis=-1, keepdims=True)
    out_ref[...] = (out - mean) * lax.rsqrt(var + 1e-5) * lng_ref[...] + lnb_ref[...]


def _finalize(n0, n1, d0, d1, bias, ln_g, ln_b):
    return pl.pallas_call(
        _fin_body,
        grid=(_BS // _ROWS_C,),
        in_specs=[
            pl.BlockSpec((_ROWS_C, _HH), lambda i: (i, 0)),
            pl.BlockSpec((_ROWS_C, _HH), lambda i: (i, 0)),
            pl.BlockSpec((_NS, _ROWS_C, 2), lambda i: (0, i, 0)),
            pl.BlockSpec((_NS, _ROWS_C, 2), lambda i: (0, i, 0)),
            pl.BlockSpec((1, _HC), lambda i: (0, 0)),
            pl.BlockSpec((1, _HC), lambda i: (0, 0)),
            pl.BlockSpec((1, _HC), lambda i: (0, 0)),
        ],
        out_specs=pl.BlockSpec((_ROWS_C, _HC), lambda i: (i, 0)),
        out_shape=jax.ShapeDtypeStruct((_BS, _HC), jnp.float32),
    )(n0, n1, d0, d1,
      bias.reshape(1, _HC), ln_g.reshape(1, _HC), ln_b.reshape(1, _HC))


def kernel(edge_index, edge_x, node_x, pe_enc, W_l, b_l, W_r, b_r, att, bias, ln_g, ln_b):
    src = edge_index[0]
    dst = edge_index[1]
    W_r1 = W_r[:_DIM]
    W_r2 = W_r[_DIM:]
    pe_pair = _pe_proj(pe_enc, W_r2, b_r)
    g_pair, hn_pair = _precompute(edge_x, node_x, W_l, W_r1, b_l)
    att_pair = att.reshape(2, _HH)
    zeros = jnp.zeros((_RPT, _HC), jnp.float32)
    zerod = jnp.zeros((_DROW,), jnp.float32)
    acc, den = _sc_edges(src, dst, pe_pair, hn_pair, g_pair, att_pair,
                         zeros, zerod)
    nums = acc[:, : _BS // 2, :].reshape(2, _BS, _HH)
    dens = den[:, :, : 2 * _BS].reshape(2, _NS, _BS, 2)
    return _finalize(nums[0], nums[1], dens[0], dens[1], bias, ln_g, ln_b)
